# Initial kernel scaffold; baseline (speedup 1.0000x reference)
import jax
import jax.numpy as jnp
from jax.experimental import pallas as pl


def kernel(x, edge_index, edge_attr, W1, a_src1, a_dst1, We1, ae1, b1, W2, a_src2, a_dst2, We2, ae2, b2, W3, a_src3, a_dst3, We3, ae3, b3):
    raise NotImplementedError("write your pallas kernel here")


# speedup vs baseline: 17.1863x; 17.1863x over previous
import functools

import jax
import jax.numpy as jnp
from jax import lax
from jax.experimental import pallas as pl
from jax.experimental.pallas import tpu as pltpu
from jax.experimental.pallas import tpu_sc as plsc

N = 10000
E = 320000
D_EDGE = 16

NC = 2
NS = 16
NW = NC * NS
BLK = 128
CHUNK = ((E + NW * BLK - 1) // (NW * BLK)) * BLK
NBLK = CHUNK // BLK
E_PAD = CHUNK * NW
NP = 10240
NPT = NP // NS
BN = 1024
BE = 512


def _leaky(v):
    return jnp.where(v >= 0.0, v, 0.2 * v)


def _elu(v):
    return jnp.where(v > 0.0, v, jnp.exp(jnp.minimum(v, 0.0)) - 1.0)


def _alpha_e_body(ea_ref, vt_ref, out_ref, mx_ref):
    i = pl.program_id(0)
    rows = vt_ref[...]
    ea = ea_ref[...]

    @pl.when(i == 0)
    def _():
        mx_ref[...] = jnp.full((3, 128), -jnp.inf, jnp.float32)

    ms = []
    for r in range(3):
        blk = jnp.sum(ea * rows[r][None, :], axis=1)
        out_ref[r, pl.ds(i * BE, BE)] = blk
        ms.append(jnp.max(blk))
    mvec = jnp.stack(ms)[:, None]
    mx_ref[...] = jnp.maximum(mx_ref[...], jnp.broadcast_to(mvec, (3, 128)))


def _alpha_e(edge_attr, vt):
    grid = (E // BE,)
    out, mx = pl.pallas_call(
        _alpha_e_body,
        grid=grid,
        in_specs=[
            pl.BlockSpec((BE, D_EDGE), lambda i: (i, 0)),
            pl.BlockSpec((3, D_EDGE), lambda i: (0, 0)),
        ],
        out_specs=[
            pl.BlockSpec((3, E), lambda i: (0, 0)),
            pl.BlockSpec((3, 128), lambda i: (0, 0)),
        ],
        out_shape=[
            jax.ShapeDtypeStruct((3, E), jnp.float32),
            jax.ShapeDtypeStruct((3, 128), jnp.float32),
        ],
    )(edge_attr, vt)
    return out, mx


def _dense_first_body(x_ref, w_ref, asv_ref, adv_ref,
                      h_ref, asrc_ref, adst_ref, mx_ref):
    i = pl.program_id(0)
    h = jnp.dot(x_ref[...], w_ref[...], preferred_element_type=jnp.float32)
    h_ref[...] = h
    a_s = jnp.sum(h * asv_ref[...], axis=1)
    a_d = jnp.sum(h * adv_ref[...], axis=1)
    asrc_ref[0, pl.ds(i * BN, BN)] = a_s
    adst_ref[0, pl.ds(i * BN, BN)] = a_d

    @pl.when(i == 0)
    def _():
        mx_ref[...] = jnp.full((2, 128), -jnp.inf, jnp.float32)

    cur = mx_ref[...]
    row0 = (lax.broadcasted_iota(jnp.int32, (2, 128), 0) == 0)
    upd = jnp.where(row0, jnp.maximum(cur, jnp.max(a_s)),
                    jnp.maximum(cur, jnp.max(a_d)))
    mx_ref[...] = upd


def _dense_mid_body(acc_ref, den_ref, b_ref, w_ref, asv_ref, adv_ref,
                    h_ref, asrc_ref, adst_ref, mx_ref):
    i = pl.program_id(0)
    den = (den_ref[0, 0, pl.ds(i * BN, BN)] +
           den_ref[1, 0, pl.ds(i * BN, BN)] + 1e-16)
    num = acc_ref[0] + acc_ref[1]
    xa = _elu(num / den[:, None] + b_ref[...])
    h = jnp.dot(xa, w_ref[...], preferred_element_type=jnp.float32)
    h_ref[...] = h
    a_s = jnp.sum(h * asv_ref[...], axis=1)
    a_d = jnp.sum(h * adv_ref[...], axis=1)
    asrc_ref[0, pl.ds(i * BN, BN)] = a_s
    adst_ref[0, pl.ds(i * BN, BN)] = a_d

    @pl.when(i == 0)
    def _():
        mx_ref[...] = jnp.full((2, 128), -jnp.inf, jnp.float32)

    cur = mx_ref[...]
    row0 = (lax.broadcasted_iota(jnp.int32, (2, 128), 0) == 0)
    upd = jnp.where(row0, jnp.maximum(cur, jnp.max(a_s)),
                    jnp.maximum(cur, jnp.max(a_d)))
    mx_ref[...] = upd


def _dense_first(x, w, a_src, a_dst, dout):
    din = x.shape[1]
    grid = (NP // BN,)
    return pl.pallas_call(
        _dense_first_body,
        grid=grid,
        in_specs=[
            pl.BlockSpec((BN, din), lambda i: (i, 0)),
            pl.BlockSpec((din, dout), lambda i: (0, 0)),
            pl.BlockSpec((1, dout), lambda i: (0, 0)),
            pl.BlockSpec((1, dout), lambda i: (0, 0)),
        ],
        out_specs=[
            pl.BlockSpec((BN, dout), lambda i: (i, 0)),
            pl.BlockSpec((1, NP), lambda i: (0, 0)),
            pl.BlockSpec((1, NP), lambda i: (0, 0)),
            pl.BlockSpec((2, 128), lambda i: (0, 0)),
        ],
        out_shape=[
            jax.ShapeDtypeStruct((NP, dout), jnp.float32),
            jax.ShapeDtypeStruct((1, NP), jnp.float32),
            jax.ShapeDtypeStruct((1, NP), jnp.float32),
            jax.ShapeDtypeStruct((2, 128), jnp.float32),
        ],
    )(x, w, a_src.reshape(1, dout), a_dst.reshape(1, dout))


def _dense_mid(acc, den, b, w, a_src, a_dst, dprev, dout):
    grid = (NP // BN,)
    return pl.pallas_call(
        _dense_mid_body,
        grid=grid,
        in_specs=[
            pl.BlockSpec((2, BN, dprev), lambda i: (0, i, 0)),
            pl.BlockSpec((2, 1, NP), lambda i: (0, 0, 0)),
            pl.BlockSpec((1, dprev), lambda i: (0, 0)),
            pl.BlockSpec((dprev, dout), lambda i: (0, 0)),
            pl.BlockSpec((1, dout), lambda i: (0, 0)),
            pl.BlockSpec((1, dout), lambda i: (0, 0)),
        ],
        out_specs=[
            pl.BlockSpec((BN, dout), lambda i: (i, 0)),
            pl.BlockSpec((1, NP), lambda i: (0, 0)),
            pl.BlockSpec((1, NP), lambda i: (0, 0)),
            pl.BlockSpec((2, 128), lambda i: (0, 0)),
        ],
        out_shape=[
            jax.ShapeDtypeStruct((NP, dout), jnp.float32),
            jax.ShapeDtypeStruct((1, NP), jnp.float32),
            jax.ShapeDtypeStruct((1, NP), jnp.float32),
            jax.ShapeDtypeStruct((2, 128), jnp.float32),
        ],
    )(acc, den.reshape(2, 1, NP), b.reshape(1, dprev), w,
      a_src.reshape(1, dout), a_dst.reshape(1, dout))


def _finish_body(acc_ref, den_ref, b_ref, z_ref):
    i = pl.program_id(0)
    den = (den_ref[0, 0, pl.ds(i * BN, BN)] +
           den_ref[1, 0, pl.ds(i * BN, BN)] + 1e-16)
    num = acc_ref[0] + acc_ref[1]
    z_ref[...] = num / den[:, None] + b_ref[...]


def _finish(acc, den, b, dout):
    grid = (NP // BN,)
    return pl.pallas_call(
        _finish_body,
        grid=grid,
        in_specs=[
            pl.BlockSpec((2, BN, dout), lambda i: (0, i, 0)),
            pl.BlockSpec((2, 1, NP), lambda i: (0, 0, 0)),
            pl.BlockSpec((1, dout), lambda i: (0, 0)),
        ],
        out_specs=pl.BlockSpec((BN, dout), lambda i: (i, 0)),
        out_shape=jax.ShapeDtypeStruct((NP, dout), jnp.float32),
    )(acc, den.reshape(2, 1, NP), b.reshape(1, dout))


@functools.lru_cache(maxsize=None)
def _sc_edge_pass(D):
    mesh = plsc.VectorSubcoreMesh(core_axis_name="c", subcore_axis_name="s",
                                  num_cores=NC, num_subcores=NS)

    @functools.partial(
        pl.kernel,
        out_type=[
            jax.ShapeDtypeStruct((NC, NP, D), jnp.float32),
            jax.ShapeDtypeStruct((NC, NP), jnp.float32),
        ],
        mesh=mesh,
        compiler_params=pltpu.CompilerParams(use_tc_tiling_on_sc=False,
                                             needs_layout_passes=False),
        scratch_types=[
            pltpu.VMEM((NP,), jnp.float32),
            pltpu.VMEM((NP,), jnp.float32),
            pltpu.VMEM((BLK,), jnp.int32),
            pltpu.VMEM((BLK,), jnp.int32),
            pltpu.VMEM((BLK,), jnp.float32),
            pltpu.VMEM((BLK,), jnp.float32),
            pltpu.VMEM((BLK, D), jnp.float32),
            pltpu.VMEM((NPT,), jnp.float32),
            pltpu.VMEM((16,), jnp.float32),
            pltpu.VMEM_SHARED((NP, D), jnp.float32),
            pltpu.VMEM_SHARED((NP,), jnp.float32),
            pltpu.SemaphoreType.DMA,
            pltpu.SemaphoreType.DMA,
            pltpu.SemaphoreType.DMA,
        ],
    )
    def kern(src_hbm, dst_hbm, ae_hbm, asrc_hbm, adst_hbm, h_hbm, bvec_hbm,
             out_hbm, den_hbm,
             asrc_v, adst_v, srcb, dstb, aeb, exb, rowb, zb, bvec_v,
             out_sh, den_sh, sem_g, sem_s, sem_d):
        cid = lax.axis_index("c")
        sid = lax.axis_index("s")
        wid = cid * NS + sid
        ebase = wid * CHUNK

        pltpu.sync_copy(asrc_hbm, asrc_v)
        pltpu.sync_copy(adst_hbm, adst_v)
        pltpu.sync_copy(bvec_hbm, bvec_v)
        bv = bvec_v[...]

        @pl.loop(0, NPT // 16)
        def _zz(i):
            zb[pl.ds(i * 16, 16)] = jnp.zeros((16,), jnp.float32)

        @pl.loop(0, BLK)
        def _zr(r):
            for kk in range(D // 16):
                rowb[r, pl.ds(kk * 16, 16)] = jnp.zeros((16,), jnp.float32)

        rbase = sid * NPT
        for i in range(5):
            pltpu.sync_copy(rowb,
                            out_sh.at[pl.ds(rbase + i * BLK, BLK)])
        pltpu.sync_copy(zb, den_sh.at[pl.ds(rbase, NPT)])

        plsc.subcore_barrier()

        @pl.loop(0, NBLK)
        def _blk(bi):
            eb = ebase + bi * BLK
            pltpu.sync_copy(src_hbm.at[pl.ds(eb, BLK)], srcb)
            pltpu.sync_copy(dst_hbm.at[pl.ds(eb, BLK)], dstb)
            pltpu.sync_copy(ae_hbm.at[pl.ds(eb, BLK)], aeb)
            gd = pltpu.async_copy(h_hbm.at[srcb], rowb, sem_g)
            for g in range(BLK // 16):
                s16 = srcb[pl.ds(g * 16, 16)]
                d16 = dstb[pl.ds(g * 16, 16)]
                a1 = plsc.load_gather(asrc_v, [s16])
                a2 = plsc.load_gather(adst_v, [d16])
                al = _leaky(a1 + a2 + aeb[pl.ds(g * 16, 16)]) - bv
                exb[pl.ds(g * 16, 16)] = jnp.exp(al)
            gd.wait()

            @pl.loop(0, BLK)
            def _row(r):
                ridx = jnp.broadcast_to(r, (16,)).astype(jnp.int32)
                exsp = plsc.load_gather(exb, [ridx])
                for kk in range(D // 16):
                    rowb[r, pl.ds(kk * 16, 16)] = (
                        rowb[r, pl.ds(kk * 16, 16)] * exsp)

            pltpu.async_copy(rowb, out_sh.at[dstb], sem_s, add=True).wait()
            pltpu.async_copy(exb, den_sh.at[dstb], sem_d, add=True).wait()

        plsc.subcore_barrier()

        for i in range(5):
            rb = rbase + i * BLK
            pltpu.sync_copy(out_sh.at[pl.ds(rb, BLK)], rowb)
            pltpu.sync_copy(rowb, out_hbm.at[cid].at[pl.ds(rb, BLK)])
        pltpu.sync_copy(den_sh.at[pl.ds(rbase, NPT)], zb)
        pltpu.sync_copy(zb, den_hbm.at[cid].at[pl.ds(rbase, NPT)])

    return kern


def kernel(x, edge_index, edge_attr,
           W1, a_src1, a_dst1, We1, ae1, b1,
           W2, a_src2, a_dst2, We2, ae2, b2,
           W3, a_src3, a_dst3, We3, ae3, b3):
    src = edge_index[0]
    dst = edge_index[1]
    pad = E_PAD - E
    src_p = jnp.pad(src, (0, pad))
    dst_p = jnp.pad(dst, (0, pad))
    x = jnp.pad(x, ((0, NP - N), (0, 0)))

    vt = jnp.stack([We1 @ ae1, We2 @ ae2, We3 @ ae3], axis=0)
    ae_all, ae_mx = _alpha_e(edge_attr, vt)
    ae_pad = jnp.pad(ae_all, ((0, 0), (0, pad)), constant_values=-1e30)

    dims = [(W1, a_src1, a_dst1, b1, 64),
            (W2, a_src2, a_dst2, b2, 32),
            (W3, a_src3, a_dst3, b3, 16)]

    acc = den = None
    dprev = None
    for li, (W, a_s, a_d, b, dout) in enumerate(dims):
        if li == 0:
            h, asrc, adst, mx = _dense_first(x, W, a_s, a_d, dout)
        else:
            bprev = dims[li - 1][3]
            h, asrc, adst, mx = _dense_mid(
                acc, den, bprev, W, a_s, a_d, dprev, dout)
        bound = jnp.maximum(mx[0, 0] + mx[1, 0] + ae_mx[li, 0], 0.0)
        bvec = jnp.broadcast_to(bound, (16,))
        acc, den = _sc_edge_pass(dout)(
            src_p, dst_p, ae_pad[li], asrc.reshape(NP), adst.reshape(NP),
            h, bvec)
        dprev = dout

    z = _finish(acc, den, b3, 16)
    return z[:N]


# speedup vs baseline: 24.9179x; 1.4499x over previous
import functools

import jax
import jax.numpy as jnp
from jax import lax
from jax.experimental import pallas as pl
from jax.experimental.pallas import tpu as pltpu
from jax.experimental.pallas import tpu_sc as plsc

N = 10000
E = 320000
D_EDGE = 16

NC = 2
NS = 16
NW = NC * NS
BLK = 128
CHUNK = ((E + NW * BLK - 1) // (NW * BLK)) * BLK
NBLK = CHUNK // BLK
E_PAD = CHUNK * NW
NP = 10240
NPT = NP // NS
BN = 1024
BE = 512


def _leaky(v):
    return jnp.where(v >= 0.0, v, 0.2 * v)


def _elu(v):
    return jnp.where(v > 0.0, v, jnp.exp(jnp.minimum(v, 0.0)) - 1.0)


def _alpha_e_body(ea_ref, vt_ref, out_ref, mx_ref):
    i = pl.program_id(0)
    rows = vt_ref[...]
    ea = ea_ref[...]

    @pl.when(i == 0)
    def _():
        mx_ref[...] = jnp.full((3, 128), -jnp.inf, jnp.float32)

    ms = []
    for r in range(3):
        blk = jnp.sum(ea * rows[r][None, :], axis=1)
        out_ref[r, pl.ds(i * BE, BE)] = blk
        ms.append(jnp.max(blk))
    mvec = jnp.stack(ms)[:, None]
    mx_ref[...] = jnp.maximum(mx_ref[...], jnp.broadcast_to(mvec, (3, 128)))


def _alpha_e(edge_attr, vt):
    grid = (E // BE,)
    out, mx = pl.pallas_call(
        _alpha_e_body,
        grid=grid,
        in_specs=[
            pl.BlockSpec((BE, D_EDGE), lambda i: (i, 0)),
            pl.BlockSpec((3, D_EDGE), lambda i: (0, 0)),
        ],
        out_specs=[
            pl.BlockSpec((3, E), lambda i: (0, 0)),
            pl.BlockSpec((3, 128), lambda i: (0, 0)),
        ],
        out_shape=[
            jax.ShapeDtypeStruct((3, E), jnp.float32),
            jax.ShapeDtypeStruct((3, 128), jnp.float32),
        ],
    )(edge_attr, vt)
    return out, mx


def _dense_first_body(x_ref, w_ref, asv_ref, adv_ref,
                      h_ref, asrc_ref, adst_ref, mx_ref):
    i = pl.program_id(0)
    h = jnp.dot(x_ref[...], w_ref[...], preferred_element_type=jnp.float32)
    h_ref[...] = h
    a_s = jnp.sum(h * asv_ref[...], axis=1)
    a_d = jnp.sum(h * adv_ref[...], axis=1)
    asrc_ref[0, pl.ds(i * BN, BN)] = a_s
    adst_ref[0, pl.ds(i * BN, BN)] = a_d

    @pl.when(i == 0)
    def _():
        mx_ref[...] = jnp.full((2, 128), -jnp.inf, jnp.float32)

    cur = mx_ref[...]
    row0 = (lax.broadcasted_iota(jnp.int32, (2, 128), 0) == 0)
    upd = jnp.where(row0, jnp.maximum(cur, jnp.max(a_s)),
                    jnp.maximum(cur, jnp.max(a_d)))
    mx_ref[...] = upd


def _dense_mid_body(acc_ref, den_ref, b_ref, w_ref, asv_ref, adv_ref,
                    h_ref, asrc_ref, adst_ref, mx_ref):
    i = pl.program_id(0)
    den = (den_ref[0, 0, pl.ds(i * BN, BN)] +
           den_ref[1, 0, pl.ds(i * BN, BN)] + 1e-16)
    num = acc_ref[0] + acc_ref[1]
    xa = _elu(num / den[:, None] + b_ref[...])
    h = jnp.dot(xa, w_ref[...], preferred_element_type=jnp.float32)
    h_ref[...] = h
    a_s = jnp.sum(h * asv_ref[...], axis=1)
    a_d = jnp.sum(h * adv_ref[...], axis=1)
    asrc_ref[0, pl.ds(i * BN, BN)] = a_s
    adst_ref[0, pl.ds(i * BN, BN)] = a_d

    @pl.when(i == 0)
    def _():
        mx_ref[...] = jnp.full((2, 128), -jnp.inf, jnp.float32)

    cur = mx_ref[...]
    row0 = (lax.broadcasted_iota(jnp.int32, (2, 128), 0) == 0)
    upd = jnp.where(row0, jnp.maximum(cur, jnp.max(a_s)),
                    jnp.maximum(cur, jnp.max(a_d)))
    mx_ref[...] = upd


def _dense_first(x, w, a_src, a_dst, dout):
    din = x.shape[1]
    grid = (NP // BN,)
    return pl.pallas_call(
        _dense_first_body,
        grid=grid,
        in_specs=[
            pl.BlockSpec((BN, din), lambda i: (i, 0)),
            pl.BlockSpec((din, dout), lambda i: (0, 0)),
            pl.BlockSpec((1, dout), lambda i: (0, 0)),
            pl.BlockSpec((1, dout), lambda i: (0, 0)),
        ],
        out_specs=[
            pl.BlockSpec((BN, dout), lambda i: (i, 0)),
            pl.BlockSpec((1, NP), lambda i: (0, 0)),
            pl.BlockSpec((1, NP), lambda i: (0, 0)),
            pl.BlockSpec((2, 128), lambda i: (0, 0)),
        ],
        out_shape=[
            jax.ShapeDtypeStruct((NP, dout), jnp.float32),
            jax.ShapeDtypeStruct((1, NP), jnp.float32),
            jax.ShapeDtypeStruct((1, NP), jnp.float32),
            jax.ShapeDtypeStruct((2, 128), jnp.float32),
        ],
    )(x, w, a_src.reshape(1, dout), a_dst.reshape(1, dout))


def _dense_mid(acc, den, b, w, a_src, a_dst, dprev, dout):
    grid = (NP // BN,)
    return pl.pallas_call(
        _dense_mid_body,
        grid=grid,
        in_specs=[
            pl.BlockSpec((2, BN, dprev), lambda i: (0, i, 0)),
            pl.BlockSpec((2, 1, NP), lambda i: (0, 0, 0)),
            pl.BlockSpec((1, dprev), lambda i: (0, 0)),
            pl.BlockSpec((dprev, dout), lambda i: (0, 0)),
            pl.BlockSpec((1, dout), lambda i: (0, 0)),
            pl.BlockSpec((1, dout), lambda i: (0, 0)),
        ],
        out_specs=[
            pl.BlockSpec((BN, dout), lambda i: (i, 0)),
            pl.BlockSpec((1, NP), lambda i: (0, 0)),
            pl.BlockSpec((1, NP), lambda i: (0, 0)),
            pl.BlockSpec((2, 128), lambda i: (0, 0)),
        ],
        out_shape=[
            jax.ShapeDtypeStruct((NP, dout), jnp.float32),
            jax.ShapeDtypeStruct((1, NP), jnp.float32),
            jax.ShapeDtypeStruct((1, NP), jnp.float32),
            jax.ShapeDtypeStruct((2, 128), jnp.float32),
        ],
    )(acc, den.reshape(2, 1, NP), b.reshape(1, dprev), w,
      a_src.reshape(1, dout), a_dst.reshape(1, dout))


def _finish_body(acc_ref, den_ref, b_ref, z_ref):
    i = pl.program_id(0)
    den = (den_ref[0, 0, pl.ds(i * BN, BN)] +
           den_ref[1, 0, pl.ds(i * BN, BN)] + 1e-16)
    num = acc_ref[0] + acc_ref[1]
    z_ref[...] = num / den[:, None] + b_ref[...]


def _finish(acc, den, b, dout):
    grid = (NP // BN,)
    return pl.pallas_call(
        _finish_body,
        grid=grid,
        in_specs=[
            pl.BlockSpec((2, BN, dout), lambda i: (0, i, 0)),
            pl.BlockSpec((2, 1, NP), lambda i: (0, 0, 0)),
            pl.BlockSpec((1, dout), lambda i: (0, 0)),
        ],
        out_specs=pl.BlockSpec((BN, dout), lambda i: (i, 0)),
        out_shape=jax.ShapeDtypeStruct((NP, dout), jnp.float32),
    )(acc, den.reshape(2, 1, NP), b.reshape(1, dout))


@functools.lru_cache(maxsize=None)
def _sc_edge_pass(D):
    mesh = plsc.VectorSubcoreMesh(core_axis_name="c", subcore_axis_name="s",
                                  num_cores=NC, num_subcores=NS)

    @functools.partial(
        pl.kernel,
        out_type=[
            jax.ShapeDtypeStruct((NC, NP, D), jnp.float32),
            jax.ShapeDtypeStruct((NC, NP), jnp.float32),
        ],
        mesh=mesh,
        compiler_params=pltpu.CompilerParams(use_tc_tiling_on_sc=False,
                                             needs_layout_passes=False),
        scratch_types=[
            pltpu.VMEM((NP,), jnp.float32),
            pltpu.VMEM((NP,), jnp.float32),
            pltpu.VMEM((CHUNK,), jnp.int32),
            pltpu.VMEM((CHUNK,), jnp.int32),
            pltpu.VMEM((CHUNK,), jnp.float32),
            pltpu.VMEM((3, BLK), jnp.float32),
            pltpu.VMEM((3, BLK), jnp.int32),
            pltpu.VMEM((3, BLK, D), jnp.float32),
            pltpu.VMEM((NPT,), jnp.float32),
            pltpu.VMEM((16,), jnp.float32),
            pltpu.VMEM_SHARED((NP, D), jnp.float32),
            pltpu.VMEM_SHARED((NP,), jnp.float32),
            pltpu.SemaphoreType.DMA,
            pltpu.SemaphoreType.DMA,
            pltpu.SemaphoreType.DMA,
            pltpu.SemaphoreType.DMA,
            pltpu.SemaphoreType.DMA,
            pltpu.SemaphoreType.DMA,
            pltpu.SemaphoreType.DMA,
            pltpu.SemaphoreType.DMA,
            pltpu.SemaphoreType.DMA,
        ],
    )
    def kern(src_hbm, dst_hbm, ae_hbm, asrc_hbm, adst_hbm, h_hbm, bvec_hbm,
             out_hbm, den_hbm,
             asrc_v, adst_v, src_v, dst_v, ae_v, exb, dstb, rowb, zb, bvec_v,
             out_sh, den_sh,
             sg0, sg1, sg2, so0, so1, so2, sd0, sd1, sd2):
        cid = lax.axis_index("c")
        sid = lax.axis_index("s")
        wid = cid * NS + sid
        ebase = wid * CHUNK
        sems_g = (sg0, sg1, sg2)
        sems_o = (so0, so1, so2)
        sems_d = (sd0, sd1, sd2)

        pltpu.sync_copy(src_hbm.at[pl.ds(ebase, CHUNK)], src_v)
        pltpu.sync_copy(dst_hbm.at[pl.ds(ebase, CHUNK)], dst_v)
        pltpu.sync_copy(ae_hbm.at[pl.ds(ebase, CHUNK)], ae_v)
        pltpu.sync_copy(asrc_hbm, asrc_v)
        pltpu.sync_copy(adst_hbm, adst_v)
        pltpu.sync_copy(bvec_hbm, bvec_v)
        bv = bvec_v[...]

        @pl.loop(0, NPT // 16)
        def _zz(i):
            zb[pl.ds(i * 16, 16)] = jnp.zeros((16,), jnp.float32)

        @pl.loop(0, BLK)
        def _zr(r):
            for kk in range(D // 16):
                rowb[0, r, pl.ds(kk * 16, 16)] = jnp.zeros((16,), jnp.float32)

        rbase = sid * NPT
        for i in range(5):
            pltpu.sync_copy(rowb.at[0],
                            out_sh.at[pl.ds(rbase + i * BLK, BLK)])
        pltpu.sync_copy(zb, den_sh.at[pl.ds(rbase, NPT)])

        plsc.subcore_barrier()

        def do_block(b, j, jn):
            @pl.when(b < NBLK)
            def _():
                boff = ebase + b * BLK

                @pl.when(b >= 2)
                def _():
                    pltpu.make_async_copy(
                        rowb.at[jn], out_sh.at[dstb.at[jn]], sems_o[jn]).wait()
                    pltpu.make_async_copy(
                        exb.at[jn], den_sh.at[dstb.at[jn]], sems_d[jn]).wait()

                @pl.when(b + 1 < NBLK)
                def _():
                    nboff = ebase + (b + 1) * BLK
                    pltpu.async_copy(
                        h_hbm.at[src_v.at[pl.ds((b + 1) * BLK, BLK)]],
                        rowb.at[jn], sems_g[jn])

                for g in range(BLK // 16):
                    lo = b * BLK + g * 16
                    s16 = src_v[pl.ds(lo, 16)]
                    d16 = dst_v[pl.ds(lo, 16)]
                    a1 = plsc.load_gather(asrc_v, [s16])
                    a2 = plsc.load_gather(adst_v, [d16])
                    al = _leaky(a1 + a2 + ae_v[pl.ds(lo, 16)]) - bv
                    exb[j, pl.ds(g * 16, 16)] = jnp.exp(al)
                    dstb[j, pl.ds(g * 16, 16)] = d16

                pltpu.make_async_copy(
                    h_hbm.at[src_v.at[pl.ds(b * BLK, BLK)]],
                    rowb.at[j], sems_g[j]).wait()

                @pl.loop(0, BLK)
                def _row(r):
                    ridx = jnp.broadcast_to(r, (16,)).astype(jnp.int32)
                    exsp = plsc.load_gather(exb.at[j], [ridx])
                    for kk in range(D // 16):
                        rowb[j, r, pl.ds(kk * 16, 16)] = (
                            rowb[j, r, pl.ds(kk * 16, 16)] * exsp)

                pltpu.async_copy(rowb.at[j], out_sh.at[dstb.at[j]],
                                 sems_o[j], add=True)
                pltpu.async_copy(exb.at[j], den_sh.at[dstb.at[j]],
                                 sems_d[j], add=True)

        pltpu.async_copy(h_hbm.at[src_v.at[pl.ds(0, BLK)]],
                         rowb.at[0], sems_g[0])

        @pl.loop(0, (NBLK + 2) // 3)
        def _trip(i):
            b0 = i * 3
            do_block(b0, 0, 1)
            do_block(b0 + 1, 1, 2)
            do_block(b0 + 2, 2, 0)

        for j in (2, 0):
            pltpu.make_async_copy(
                rowb.at[j], out_sh.at[dstb.at[j]], sems_o[j]).wait()
            pltpu.make_async_copy(
                exb.at[j], den_sh.at[dstb.at[j]], sems_d[j]).wait()

        plsc.subcore_barrier()

        for i in range(5):
            rb = rbase + i * BLK
            pltpu.sync_copy(out_sh.at[pl.ds(rb, BLK)], rowb.at[0])
            pltpu.sync_copy(rowb.at[0], out_hbm.at[cid].at[pl.ds(rb, BLK)])
        pltpu.sync_copy(den_sh.at[pl.ds(rbase, NPT)], zb)
        pltpu.sync_copy(zb, den_hbm.at[cid].at[pl.ds(rbase, NPT)])

    return kern


def kernel(x, edge_index, edge_attr,
           W1, a_src1, a_dst1, We1, ae1, b1,
           W2, a_src2, a_dst2, We2, ae2, b2,
           W3, a_src3, a_dst3, We3, ae3, b3):
    src = edge_index[0]
    dst = edge_index[1]
    pad = E_PAD - E
    src_p = jnp.pad(src, (0, pad))
    dst_p = jnp.pad(dst, (0, pad))
    x = jnp.pad(x, ((0, NP - N), (0, 0)))

    vt = jnp.stack([We1 @ ae1, We2 @ ae2, We3 @ ae3], axis=0)
    ae_all, ae_mx = _alpha_e(edge_attr, vt)
    ae_pad = jnp.pad(ae_all, ((0, 0), (0, pad)), constant_values=-1e30)

    dims = [(W1, a_src1, a_dst1, b1, 64),
            (W2, a_src2, a_dst2, b2, 32),
            (W3, a_src3, a_dst3, b3, 16)]

    acc = den = None
    dprev = None
    for li, (W, a_s, a_d, b, dout) in enumerate(dims):
        if li == 0:
            h, asrc, adst, mx = _dense_first(x, W, a_s, a_d, dout)
        else:
            bprev = dims[li - 1][3]
            h, asrc, adst, mx = _dense_mid(
                acc, den, bprev, W, a_s, a_d, dprev, dout)
        bound = jnp.maximum(mx[0, 0] + mx[1, 0] + ae_mx[li, 0], 0.0)
        bvec = jnp.broadcast_to(bound, (16,))
        acc, den = _sc_edge_pass(dout)(
            src_p, dst_p, ae_pad[li], asrc.reshape(NP), adst.reshape(NP),
            h, bvec)
        dprev = dout

    z = _finish(acc, den, b3, 16)
    return z[:N]


# speedup vs baseline: 27.3515x; 1.0977x over previous
import functools

import jax
import jax.numpy as jnp
from jax import lax
from jax.experimental import pallas as pl
from jax.experimental.pallas import tpu as pltpu
from jax.experimental.pallas import tpu_sc as plsc

N = 10000
E = 320000
D_EDGE = 16

NC = 2
NS = 16
NW = NC * NS
BLK = 80
CHUNK = E // NW
NBLK = CHUNK // BLK
NP = 10240
NPT = NP // NS
BN = 1024
BE = 512


def _leaky(v):
    return jnp.where(v >= 0.0, v, 0.2 * v)


def _elu(v):
    return jnp.where(v > 0.0, v, jnp.exp(jnp.minimum(v, 0.0)) - 1.0)


def _alpha_e_body(ea_ref, we1_ref, ae1_ref, we2_ref, ae2_ref,
                  we3_ref, ae3_ref, out_ref, mx_ref):
    i = pl.program_id(0)
    v1 = jnp.sum(we1_ref[...] * ae1_ref[...], axis=1)
    v2 = jnp.sum(we2_ref[...] * ae2_ref[...], axis=1)
    v3 = jnp.sum(we3_ref[...] * ae3_ref[...], axis=1)
    rows = jnp.stack([v1, v2, v3], axis=0)
    ea = ea_ref[...]

    @pl.when(i == 0)
    def _():
        mx_ref[...] = jnp.full((3, 128), -jnp.inf, jnp.float32)

    ms = []
    for r in range(3):
        blk = jnp.sum(ea * rows[r][None, :], axis=1)
        out_ref[r, pl.ds(i * BE, BE)] = blk
        ms.append(jnp.max(blk))
    mvec = jnp.stack(ms)[:, None]
    mx_ref[...] = jnp.maximum(mx_ref[...], jnp.broadcast_to(mvec, (3, 128)))


def _alpha_e(edge_attr, we1, ae1, we2, ae2, we3, ae3):
    grid = (E // BE,)
    wspec = lambda d: pl.BlockSpec((D_EDGE, d), lambda i: (0, 0))
    aspec = lambda d: pl.BlockSpec((1, d), lambda i: (0, 0))
    out, mx = pl.pallas_call(
        _alpha_e_body,
        grid=grid,
        in_specs=[
            pl.BlockSpec((BE, D_EDGE), lambda i: (i, 0)),
            wspec(64), aspec(64), wspec(32), aspec(32), wspec(16), aspec(16),
        ],
        out_specs=[
            pl.BlockSpec((3, E), lambda i: (0, 0)),
            pl.BlockSpec((3, 128), lambda i: (0, 0)),
        ],
        out_shape=[
            jax.ShapeDtypeStruct((3, E), jnp.float32),
            jax.ShapeDtypeStruct((3, 128), jnp.float32),
        ],
    )(edge_attr, we1, ae1.reshape(1, 64), we2, ae2.reshape(1, 32),
      we3, ae3.reshape(1, 16))
    return out, mx


def _dense_first_body(x_ref, w_ref, asv_ref, adv_ref, aemx_ref,
                      h_ref, asrc_ref, adst_ref, mx_ref, bv_ref, *, li):
    i = pl.program_id(0)
    h = jnp.dot(x_ref[...], w_ref[...], preferred_element_type=jnp.float32)
    h_ref[...] = h
    a_s = jnp.sum(h * asv_ref[...], axis=1)
    a_d = jnp.sum(h * adv_ref[...], axis=1)
    asrc_ref[0, pl.ds(i * BN, BN)] = a_s
    adst_ref[0, pl.ds(i * BN, BN)] = a_d

    @pl.when(i == 0)
    def _():
        mx_ref[...] = jnp.full((2, 128), -jnp.inf, jnp.float32)

    cur = mx_ref[...]
    row0 = (lax.broadcasted_iota(jnp.int32, (2, 128), 0) == 0)
    upd = jnp.where(row0, jnp.maximum(cur, jnp.max(a_s)),
                    jnp.maximum(cur, jnp.max(a_d)))
    mx_ref[...] = upd

    @pl.when(i == NP // BN - 1)
    def _():
        bound = jnp.maximum(upd[0, 0] + upd[1, 0] + aemx_ref[li, 0], 0.0)
        bv_ref[...] = jnp.full((1, 128), bound, jnp.float32)


def _dense_mid_body(acc_ref, den_ref, b_ref, w_ref, asv_ref, adv_ref,
                    aemx_ref, h_ref, asrc_ref, adst_ref, mx_ref, bv_ref,
                    *, li):
    i = pl.program_id(0)
    den = (den_ref[0, 0, pl.ds(i * BN, BN)] +
           den_ref[1, 0, pl.ds(i * BN, BN)] + 1e-16)
    num = acc_ref[0] + acc_ref[1]
    xa = _elu(num / den[:, None] + b_ref[...])
    h = jnp.dot(xa, w_ref[...], preferred_element_type=jnp.float32)
    h_ref[...] = h
    a_s = jnp.sum(h * asv_ref[...], axis=1)
    a_d = jnp.sum(h * adv_ref[...], axis=1)
    asrc_ref[0, pl.ds(i * BN, BN)] = a_s
    adst_ref[0, pl.ds(i * BN, BN)] = a_d

    @pl.when(i == 0)
    def _():
        mx_ref[...] = jnp.full((2, 128), -jnp.inf, jnp.float32)

    cur = mx_ref[...]
    row0 = (lax.broadcasted_iota(jnp.int32, (2, 128), 0) == 0)
    upd = jnp.where(row0, jnp.maximum(cur, jnp.max(a_s)),
                    jnp.maximum(cur, jnp.max(a_d)))
    mx_ref[...] = upd

    @pl.when(i == NP // BN - 1)
    def _():
        bound = jnp.maximum(upd[0, 0] + upd[1, 0] + aemx_ref[li, 0], 0.0)
        bv_ref[...] = jnp.full((1, 128), bound, jnp.float32)


def _dense_first(x, w, a_src, a_dst, aemx, dout, li):
    din = x.shape[1]
    grid = (NP // BN,)
    return pl.pallas_call(
        functools.partial(_dense_first_body, li=li),
        grid=grid,
        in_specs=[
            pl.BlockSpec((BN, din), lambda i: (i, 0)),
            pl.BlockSpec((din, dout), lambda i: (0, 0)),
            pl.BlockSpec((1, dout), lambda i: (0, 0)),
            pl.BlockSpec((1, dout), lambda i: (0, 0)),
            pl.BlockSpec((3, 128), lambda i: (0, 0)),
        ],
        out_specs=[
            pl.BlockSpec((BN, dout), lambda i: (i, 0)),
            pl.BlockSpec((1, NP), lambda i: (0, 0)),
            pl.BlockSpec((1, NP), lambda i: (0, 0)),
            pl.BlockSpec((2, 128), lambda i: (0, 0)),
            pl.BlockSpec((1, 128), lambda i: (0, 0)),
        ],
        out_shape=[
            jax.ShapeDtypeStruct((NP, dout), jnp.float32),
            jax.ShapeDtypeStruct((1, NP), jnp.float32),
            jax.ShapeDtypeStruct((1, NP), jnp.float32),
            jax.ShapeDtypeStruct((2, 128), jnp.float32),
            jax.ShapeDtypeStruct((1, 128), jnp.float32),
        ],
    )(x, w, a_src.reshape(1, dout), a_dst.reshape(1, dout), aemx)


def _dense_mid(acc, den, b, w, a_src, a_dst, aemx, dprev, dout, li):
    grid = (NP // BN,)
    return pl.pallas_call(
        functools.partial(_dense_mid_body, li=li),
        grid=grid,
        in_specs=[
            pl.BlockSpec((2, BN, dprev), lambda i: (0, i, 0)),
            pl.BlockSpec((2, 1, NP), lambda i: (0, 0, 0)),
            pl.BlockSpec((1, dprev), lambda i: (0, 0)),
            pl.BlockSpec((dprev, dout), lambda i: (0, 0)),
            pl.BlockSpec((1, dout), lambda i: (0, 0)),
            pl.BlockSpec((1, dout), lambda i: (0, 0)),
            pl.BlockSpec((3, 128), lambda i: (0, 0)),
        ],
        out_specs=[
            pl.BlockSpec((BN, dout), lambda i: (i, 0)),
            pl.BlockSpec((1, NP), lambda i: (0, 0)),
            pl.BlockSpec((1, NP), lambda i: (0, 0)),
            pl.BlockSpec((2, 128), lambda i: (0, 0)),
            pl.BlockSpec((1, 128), lambda i: (0, 0)),
        ],
        out_shape=[
            jax.ShapeDtypeStruct((NP, dout), jnp.float32),
            jax.ShapeDtypeStruct((1, NP), jnp.float32),
            jax.ShapeDtypeStruct((1, NP), jnp.float32),
            jax.ShapeDtypeStruct((2, 128), jnp.float32),
            jax.ShapeDtypeStruct((1, 128), jnp.float32),
        ],
    )(acc, den.reshape(2, 1, NP), b.reshape(1, dprev), w,
      a_src.reshape(1, dout), a_dst.reshape(1, dout), aemx)


def _finish_body(acc_ref, den_ref, b_ref, z_ref):
    i = pl.program_id(0)
    den = (den_ref[0, 0, pl.ds(i * BN, BN)] +
           den_ref[1, 0, pl.ds(i * BN, BN)] + 1e-16)
    num = acc_ref[0] + acc_ref[1]
    z_ref[...] = num / den[:, None] + b_ref[...]


def _finish(acc, den, b, dout):
    grid = (NP // BN,)
    return pl.pallas_call(
        _finish_body,
        grid=grid,
        in_specs=[
            pl.BlockSpec((2, BN, dout), lambda i: (0, i, 0)),
            pl.BlockSpec((2, 1, NP), lambda i: (0, 0, 0)),
            pl.BlockSpec((1, dout), lambda i: (0, 0)),
        ],
        out_specs=pl.BlockSpec((BN, dout), lambda i: (i, 0)),
        out_shape=jax.ShapeDtypeStruct((NP, dout), jnp.float32),
    )(acc, den.reshape(2, 1, NP), b.reshape(1, dout))


@functools.lru_cache(maxsize=None)
def _sc_edge_pass(D, li):
    mesh = plsc.VectorSubcoreMesh(core_axis_name="c", subcore_axis_name="s",
                                  num_cores=NC, num_subcores=NS)

    @functools.partial(
        pl.kernel,
        out_type=[
            jax.ShapeDtypeStruct((NC, NP, D), jnp.float32),
            jax.ShapeDtypeStruct((NC, NP), jnp.float32),
        ],
        mesh=mesh,
        compiler_params=pltpu.CompilerParams(use_tc_tiling_on_sc=False,
                                             needs_layout_passes=False),
        scratch_types=[
            pltpu.VMEM((NP,), jnp.float32),
            pltpu.VMEM((NP,), jnp.float32),
            pltpu.VMEM((CHUNK,), jnp.int32),
            pltpu.VMEM((CHUNK,), jnp.int32),
            pltpu.VMEM((CHUNK,), jnp.float32),
            pltpu.VMEM((3, BLK), jnp.float32),
            pltpu.VMEM((3, BLK), jnp.int32),
            pltpu.VMEM((3, BLK, D), jnp.float32),
            pltpu.VMEM((NPT,), jnp.float32),
            pltpu.VMEM((16,), jnp.float32),
            pltpu.VMEM_SHARED((NP, D), jnp.float32),
            pltpu.VMEM_SHARED((NP,), jnp.float32),
            pltpu.SemaphoreType.DMA,
            pltpu.SemaphoreType.DMA,
            pltpu.SemaphoreType.DMA,
            pltpu.SemaphoreType.DMA,
            pltpu.SemaphoreType.DMA,
            pltpu.SemaphoreType.DMA,
            pltpu.SemaphoreType.DMA,
            pltpu.SemaphoreType.DMA,
            pltpu.SemaphoreType.DMA,
        ],
    )
    def kern(src_hbm, dst_hbm, ae_hbm, asrc_hbm, adst_hbm, h_hbm, bvec_hbm,
             out_hbm, den_hbm,
             asrc_v, adst_v, src_v, dst_v, ae_v, exb, dstb, rowb, zb, bvec_v,
             out_sh, den_sh,
             sg0, sg1, sg2, so0, so1, so2, sd0, sd1, sd2):
        cid = lax.axis_index("c")
        sid = lax.axis_index("s")
        wid = cid * NS + sid
        ebase = wid * CHUNK
        sems_g = (sg0, sg1, sg2)
        sems_o = (so0, so1, so2)
        sems_d = (sd0, sd1, sd2)

        pltpu.sync_copy(src_hbm.at[pl.ds(ebase, CHUNK)], src_v)
        pltpu.sync_copy(dst_hbm.at[pl.ds(ebase, CHUNK)], dst_v)
        pltpu.sync_copy(ae_hbm.at[li].at[pl.ds(ebase, CHUNK)], ae_v)
        pltpu.sync_copy(asrc_hbm, asrc_v)
        pltpu.sync_copy(adst_hbm, adst_v)
        pltpu.sync_copy(bvec_hbm.at[pl.ds(0, 16)], bvec_v)
        bv = bvec_v[...]

        @pl.loop(0, NPT // 16)
        def _zz(i):
            zb[pl.ds(i * 16, 16)] = jnp.zeros((16,), jnp.float32)

        @pl.loop(0, BLK)
        def _zr(r):
            for kk in range(D // 16):
                rowb[0, r, pl.ds(kk * 16, 16)] = jnp.zeros((16,), jnp.float32)

        rbase = sid * NPT
        for i in range(NPT // BLK):
            pltpu.sync_copy(rowb.at[0],
                            out_sh.at[pl.ds(rbase + i * BLK, BLK)])
        pltpu.sync_copy(zb, den_sh.at[pl.ds(rbase, NPT)])

        plsc.subcore_barrier()

        def do_block(b, j, jn):
            @pl.when(b < NBLK)
            def _():
                boff = ebase + b * BLK

                @pl.when(b >= 2)
                def _():
                    pltpu.make_async_copy(
                        rowb.at[jn], out_sh.at[dstb.at[jn]], sems_o[jn]).wait()
                    pltpu.make_async_copy(
                        exb.at[jn], den_sh.at[dstb.at[jn]], sems_d[jn]).wait()

                @pl.when(b + 1 < NBLK)
                def _():
                    nboff = ebase + (b + 1) * BLK
                    pltpu.async_copy(
                        h_hbm.at[src_v.at[pl.ds((b + 1) * BLK, BLK)]],
                        rowb.at[jn], sems_g[jn])

                for g in range(BLK // 16):
                    lo = b * BLK + g * 16
                    s16 = src_v[pl.ds(lo, 16)]
                    d16 = dst_v[pl.ds(lo, 16)]
                    a1 = plsc.load_gather(asrc_v, [s16])
                    a2 = plsc.load_gather(adst_v, [d16])
                    al = _leaky(a1 + a2 + ae_v[pl.ds(lo, 16)]) - bv
                    exb[j, pl.ds(g * 16, 16)] = jnp.exp(al)
                    dstb[j, pl.ds(g * 16, 16)] = d16

                pltpu.make_async_copy(
                    h_hbm.at[src_v.at[pl.ds(b * BLK, BLK)]],
                    rowb.at[j], sems_g[j]).wait()

                @pl.loop(0, BLK)
                def _row(r):
                    ridx = jnp.broadcast_to(r, (16,)).astype(jnp.int32)
                    exsp = plsc.load_gather(exb.at[j], [ridx])
                    for kk in range(D // 16):
                        rowb[j, r, pl.ds(kk * 16, 16)] = (
                            rowb[j, r, pl.ds(kk * 16, 16)] * exsp)

                pltpu.async_copy(rowb.at[j], out_sh.at[dstb.at[j]],
                                 sems_o[j], add=True)
                pltpu.async_copy(exb.at[j], den_sh.at[dstb.at[j]],
                                 sems_d[j], add=True)

        pltpu.async_copy(h_hbm.at[src_v.at[pl.ds(0, BLK)]],
                         rowb.at[0], sems_g[0])

        @pl.loop(0, (NBLK + 2) // 3)
        def _trip(i):
            b0 = i * 3
            do_block(b0, 0, 1)
            do_block(b0 + 1, 1, 2)
            do_block(b0 + 2, 2, 0)

        for j in ((NBLK - 2) % 3, (NBLK - 1) % 3):
            pltpu.make_async_copy(
                rowb.at[j], out_sh.at[dstb.at[j]], sems_o[j]).wait()
            pltpu.make_async_copy(
                exb.at[j], den_sh.at[dstb.at[j]], sems_d[j]).wait()

        plsc.subcore_barrier()

        for i in range(NPT // BLK):
            rb = rbase + i * BLK
            pltpu.sync_copy(out_sh.at[pl.ds(rb, BLK)], rowb.at[0])
            pltpu.sync_copy(rowb.at[0], out_hbm.at[cid].at[pl.ds(rb, BLK)])
        pltpu.sync_copy(den_sh.at[pl.ds(rbase, NPT)], zb)
        pltpu.sync_copy(zb, den_hbm.at[cid].at[pl.ds(rbase, NPT)])

    return kern


def kernel(x, edge_index, edge_attr,
           W1, a_src1, a_dst1, We1, ae1, b1,
           W2, a_src2, a_dst2, We2, ae2, b2,
           W3, a_src3, a_dst3, We3, ae3, b3):
    src = edge_index[0]
    dst = edge_index[1]
    x = jnp.pad(x, ((0, NP - N), (0, 0)))

    ae_all, ae_mx = _alpha_e(edge_attr, We1, ae1, We2, ae2, We3, ae3)

    dims = [(W1, a_src1, a_dst1, b1, 64),
            (W2, a_src2, a_dst2, b2, 32),
            (W3, a_src3, a_dst3, b3, 16)]

    acc = den = None
    dprev = None
    for li, (W, a_s, a_d, b, dout) in enumerate(dims):
        if li == 0:
            h, asrc, adst, mx, bv = _dense_first(x, W, a_s, a_d, ae_mx,
                                                 dout, li)
        else:
            bprev = dims[li - 1][3]
            h, asrc, adst, mx, bv = _dense_mid(
                acc, den, bprev, W, a_s, a_d, ae_mx, dprev, dout, li)
        acc, den = _sc_edge_pass(dout, li)(
            src, dst, ae_all, asrc.reshape(NP), adst.reshape(NP),
            h, bv.reshape(128))
        dprev = dout

    z = _finish(acc, den, b3, 16)
    return z[:N]
